# baseline (speedup 1.0000x reference)
import functools

import jax
import jax.numpy as jnp
from jax.experimental import pallas as pl
from jax.experimental.pallas import tpu as pltpu

_K = 16
_NG = 512
_GS = 16
_NC = _K * _GS
_MIN_SIGMA = 1e-4
_BIG = 3.0e38


def _extract_onehots(vals, iota, steps, *, take_max):
    nrows = vals.shape[0]
    onehots = []
    for _ in range(steps):
        if take_max:
            vext = jnp.max(vals, axis=0, keepdims=True)
        else:
            vext = jnp.min(vals, axis=0, keepdims=True)
        idxm = jnp.where(vals == vext, iota, nrows)
        jmin = jnp.min(idxm, axis=0, keepdims=True)
        oh = idxm == jmin
        onehots.append(oh)
        vals = jnp.where(oh, -_BIG if take_max else _BIG, vals)
    return onehots


def _bf(x):
    return x.astype(jnp.bfloat16).astype(jnp.float32)


def _soft_proj_block(paug_ref, qaug_ref, pg_ref, sig_ref, out_ref, *, n, bm):
    paug = paug_ref[0]
    qaug = qaug_ref[0]
    pg = pg_ref[0]
    inv_sigma = 1.0 / (sig_ref[0, 0] + 1e-8)

    zt = jax.lax.dot_general(
        paug, qaug, (((0,), (0,)), ((), ())),
        preferred_element_type=jnp.float32)

    g = zt[0:_NG, :]
    for e in range(1, _GS):
        g = jnp.maximum(g, zt[e * _NG:(e + 1) * _NG, :])

    iota_g = jax.lax.broadcasted_iota(jnp.int32, (_NG, bm), 0)
    onehots = _extract_onehots(g, iota_g, _K, take_max=True)
    ohcat = jnp.concatenate([oh.astype(jnp.bfloat16) for oh in onehots],
                            axis=1)

    xparts = jax.lax.dot_general(
        pg, ohcat, (((0,), (0,)), ((), ())),
        preferred_element_type=jnp.float32)

    xcoord = (xparts[96:144, :] + xparts[48:96, :]) + xparts[0:48, :]
    hps = xparts[144:192, :]

    def _cand(rows):
        return jnp.concatenate(
            [rows[:, j * bm:(j + 1) * bm] for j in range(_K)], axis=0)

    xx = _cand(xcoord[0:16, :])
    xy = _cand(xcoord[16:32, :])
    xz = _cand(xcoord[32:48, :])
    p2c = 2.0 * ((_cand(hps[32:48, :]) + _cand(hps[16:32, :]))
                 + _cand(hps[0:16, :]))

    qx = qaug[0:1, :]
    qy = qaug[1:2, :]
    qz = qaug[2:3, :]
    q2 = (qx * qx + qy * qy) + qz * qz
    innerb = (_bf(qx) * _bf(xx) + _bf(qy) * _bf(xy)) + _bf(qz) * _bf(xz)
    cd = (q2 - 2.0 * innerb) + p2c

    iota_c = jax.lax.broadcasted_iota(jnp.int32, (_NC, bm), 0)
    sel_onehots = _extract_onehots(cd, iota_c, _K, take_max=False)
    sel = sel_onehots[0]
    for oh in sel_onehots[1:]:
        sel = jnp.logical_or(sel, oh)

    dx = xx - qx
    dy = xy - qy
    dz = xz - qz
    ed = (dx * dx + dy * dy) + dz * dz

    dmin = jnp.min(jnp.where(sel, ed, _BIG), axis=0, keepdims=True)
    w = jnp.where(sel, jnp.exp((dmin - ed) * inv_sigma), 0.0)
    denom = jnp.sum(w, axis=0, keepdims=True)
    ox = jnp.sum(w * xx, axis=0, keepdims=True) / denom
    oy = jnp.sum(w * xy, axis=0, keepdims=True) / denom
    oz = jnp.sum(w * xz, axis=0, keepdims=True) / denom
    out_ref[0] = jnp.concatenate([ox, oy, oz], axis=0)


def _bf16_triple(x):
    hi = x.astype(jnp.bfloat16)
    r1 = x - hi.astype(jnp.float32)
    mid = r1.astype(jnp.bfloat16)
    lo = (r1 - mid.astype(jnp.float32)).astype(jnp.bfloat16)
    return hi, mid, lo


def kernel(point_cloud, query_cloud, temperature):
    b, c, n = point_cloud.shape
    _, _, m = query_cloud.shape
    bm = 256
    sigma = jnp.maximum(temperature * temperature, jnp.float32(_MIN_SIGMA))
    sigma = jnp.reshape(sigma, (1, 1)).astype(jnp.float32)

    hp = 0.5 * jnp.sum(point_cloud * point_cloud, axis=1)
    hh, hm, hl = _bf16_triple(hp)
    paug = jnp.concatenate(
        [point_cloud,
         -hh.astype(jnp.float32)[:, None, :],
         -hm.astype(jnp.float32)[:, None, :],
         -hl.astype(jnp.float32)[:, None, :]], axis=1)
    qaug = jnp.concatenate(
        [query_cloud, jnp.ones((b, 3, m), jnp.float32)], axis=1)

    def _group(x):
        return jnp.transpose(jnp.reshape(x, (b, _GS, _NG)), (0, 2, 1))

    trips = [_bf16_triple(point_cloud[:, cc, :]) for cc in range(c)]
    cols = [trips[0][0], trips[1][0], trips[2][0],
            trips[0][1], trips[1][1], trips[2][1],
            trips[0][2], trips[1][2], trips[2][2],
            hh, hm, hl]
    pg = jnp.concatenate(
        [_group(t.astype(jnp.float32)).astype(jnp.bfloat16)
         for t in cols], axis=2)

    grid = (b, m // bm)
    return pl.pallas_call(
        functools.partial(_soft_proj_block, n=n, bm=bm),
        grid=grid,
        in_specs=[
            pl.BlockSpec((1, 2 * c, n), lambda i, j: (i, 0, 0)),
            pl.BlockSpec((1, 2 * c, bm), lambda i, j: (i, 0, j)),
            pl.BlockSpec((1, _NG, 4 * c * _GS), lambda i, j: (i, 0, 0)),
            pl.BlockSpec(memory_space=pltpu.SMEM),
        ],
        out_specs=pl.BlockSpec((1, c, bm), lambda i, j: (i, 0, j)),
        out_shape=jax.ShapeDtypeStruct((b, c, m), jnp.float32),
    )(paug, qaug, pg, sigma)
